# Initial kernel scaffold; baseline (speedup 1.0000x reference)
import jax
import jax.numpy as jnp
from jax.experimental import pallas as pl


def kernel(flatten, features, pfcnetworks, mdcbcnetworks, v1cnetworks, shanetworks, pfcnetworkweights, mdcbcnetworkweights, v1cnetworkweights, shanetworkweights, pfc_W1, pfc_b1, pfc_g, pfc_bt, pfc_W2, pfc_b2, mdcbc_W1, mdcbc_b1, mdcbc_g, mdcbc_bt, mdcbc_W2, mdcbc_b2, v1c_W1, v1c_b1, v1c_g, v1c_bt, v1c_W2, v1c_b2, sha_W1, sha_b1, sha_g, sha_bt, sha_W2, sha_b2, gate_W, gate_b):
    raise NotImplementedError("write your pallas kernel here")


# speedup vs baseline: 27.6347x; 27.6347x over previous
import functools

import jax
import jax.numpy as jnp
from jax import lax
from jax.experimental import pallas as pl
from jax.experimental.pallas import tpu as pltpu
from jax.experimental.pallas import tpu_sc as plsc

N = 50000
E = 1600000
H = 16
NC = 2
NS = 16
NW = NC * NS
N_PAD = 51200
NPT = N_PAD // NS
RW = 400
E_PAD = NW * RW * 128
CHR = 40
NCHUNK = RW // CHR
CHP = 16
NCHUNKP = RW // CHP
ZROW = 400

_mesh = plsc.VectorSubcoreMesh(core_axis_name="c", subcore_axis_name="s")


def _zero_fill(zbuf, rows, width):
    @pl.loop(0, rows, unroll=8)
    def _(i):
        zbuf[i, :] = jnp.zeros((width,), jnp.float32)


@functools.partial(
    pl.kernel,
    out_type=jax.ShapeDtypeStruct((NC * 4 * N_PAD,), jnp.float32),
    mesh=_mesh,
    compiler_params=pltpu.CompilerParams(use_tc_tiling_on_sc=False),
    scratch_types=[
        pltpu.VMEM((CHR, 128), jnp.int32),
        pltpu.VMEM((CHR, 128), jnp.float32),
        pltpu.VMEM((NPT,), jnp.float32),
        pltpu.VMEM_SHARED((N_PAD,), jnp.float32),
        pltpu.VMEM_SHARED((N_PAD,), jnp.float32),
        pltpu.VMEM_SHARED((N_PAD,), jnp.float32),
        pltpu.VMEM_SHARED((N_PAD,), jnp.float32),
        pltpu.SemaphoreType.DMA,
    ],
)
def _deg_pass(dst0, dst1, dst2, dst3, ew0, ew1, ew2, ew3, out,
              dstv, ewv, zbuf, acc0, acc1, acc2, acc3, sem):
    cid = lax.axis_index("c")
    sid = lax.axis_index("s")
    wid = sid * NC + cid
    dsts = [dst0, dst1, dst2, dst3]
    ews = [ew0, ew1, ew2, ew3]
    accs = [acc0, acc1, acc2, acc3]
    @pl.loop(0, NPT // 16, unroll=8)
    def _(i):
        zbuf[pl.ds(i * 16, 16)] = jnp.zeros((16,), jnp.float32)

    for g in range(4):
        pltpu.sync_copy(zbuf, accs[g].at[pl.ds(sid * NPT, NPT)])
    plsc.subcore_barrier()
    for g in range(4):
        @pl.loop(0, NCHUNK)
        def _(c):
            r0 = pl.multiple_of(wid * RW + c * CHR, 8)
            pltpu.sync_copy(dsts[g].at[pl.ds(r0, CHR), :], dstv)
            pltpu.sync_copy(ews[g].at[pl.ds(r0, CHR), :], ewv)
            descs = [
                pltpu.async_copy(ewv.at[j], accs[g].at[dstv.at[j]], sem,
                                 add=True)
                for j in range(CHR)
            ]
            for d in descs:
                d.wait()
    plsc.subcore_barrier()
    for g in range(4):
        off = pl.multiple_of((cid * 4 + g) * N_PAD + sid * NPT, 128)
        pltpu.sync_copy(accs[g].at[pl.ds(sid * NPT, NPT)],
                        out.at[pl.ds(off, NPT)])


@functools.partial(
    pl.kernel,
    out_type=jax.ShapeDtypeStruct((NC * 4 * N_PAD, H), jnp.float32),
    mesh=_mesh,
    compiler_params=pltpu.CompilerParams(use_tc_tiling_on_sc=False),
    scratch_types=[
        pltpu.VMEM((CHP, 128), jnp.int32),
        pltpu.VMEM((CHP, 128), jnp.int32),
        pltpu.VMEM((CHP, 128), jnp.float32),
        pltpu.VMEM((CHP, 128, H), jnp.float32),
        pltpu.VMEM((ZROW, H), jnp.float32),
        pltpu.VMEM_SHARED((N_PAD, H), jnp.float32),
        pltpu.SemaphoreType.DMA,
    ],
)
def _prop_pass(y0, y1, y2, y3,
               src0, src1, src2, src3,
               dst0, dst1, dst2, dst3,
               ew0, ew1, ew2, ew3, out,
               srcv, dstv, ewv, rows, zbuf, acc, sem):
    cid = lax.axis_index("c")
    sid = lax.axis_index("s")
    wid = sid * NC + cid
    ys = [y0, y1, y2, y3]
    srcs = [src0, src1, src2, src3]
    dsts = [dst0, dst1, dst2, dst3]
    ews = [ew0, ew1, ew2, ew3]
    _zero_fill(zbuf, ZROW, H)

    def zero_acc():
        for k in range(NPT // ZROW):
            pltpu.sync_copy(zbuf, acc.at[pl.ds(sid * NPT + k * ZROW, ZROW), :])

    zero_acc()
    plsc.subcore_barrier()
    for g in range(4):
        @pl.loop(0, NCHUNKP)
        def _(c):
            r0 = pl.multiple_of(wid * RW + c * CHP, 8)
            pltpu.sync_copy(srcs[g].at[pl.ds(r0, CHP), :], srcv)
            pltpu.sync_copy(dsts[g].at[pl.ds(r0, CHP), :], dstv)
            pltpu.sync_copy(ews[g].at[pl.ds(r0, CHP), :], ewv)
            gd = [pltpu.async_copy(ys[g].at[srcv.at[j]], rows.at[j], sem)
                  for j in range(CHP)]
            for d in gd:
                d.wait()
            @pl.loop(0, CHP)
            def _(r):
                for l0 in range(0, 128, 16):
                    ew16 = ewv[r, pl.ds(l0, 16)]
                    for k in range(16):
                        rows[r, l0 + k, :] = rows[r, l0 + k, :] * ew16[k]
            sd = [pltpu.async_copy(rows.at[j], acc.at[dstv.at[j]], sem,
                                   add=True)
                  for j in range(CHP)]
            for d in sd:
                d.wait()
        plsc.subcore_barrier()
        roff = pl.multiple_of((cid * 4 + g) * N_PAD + sid * NPT, 128)
        pltpu.sync_copy(acc.at[pl.ds(sid * NPT, NPT), :],
                        out.at[pl.ds(roff, NPT), :])
        if g < 3:
            zero_acc()
            plsc.subcore_barrier()


def _pad_edges(ei, ew):
    src = jnp.pad(ei[0], (0, E_PAD - E)).reshape(E_PAD // 128, 128)
    dst = jnp.pad(ei[1], (0, E_PAD - E)).reshape(E_PAD // 128, 128)
    eww = jnp.pad(ew, (0, E_PAD - E)).reshape(E_PAD // 128, 128)
    return src, dst, eww


def kernel(flatten, features, pfcnetworks, mdcbcnetworks, v1cnetworks,
           shanetworks, pfcnetworkweights, mdcbcnetworkweights,
           v1cnetworkweights, shanetworkweights,
           pfc_W1, pfc_b1, pfc_g, pfc_bt, pfc_W2, pfc_b2,
           mdcbc_W1, mdcbc_b1, mdcbc_g, mdcbc_bt, mdcbc_W2, mdcbc_b2,
           v1c_W1, v1c_b1, v1c_g, v1c_bt, v1c_W2, v1c_b2,
           sha_W1, sha_b1, sha_g, sha_bt, sha_W2, sha_b2,
           gate_W, gate_b):
    eis = [pfcnetworks, mdcbcnetworks, v1cnetworks, shanetworks]
    ews_in = [pfcnetworkweights, mdcbcnetworkweights, v1cnetworkweights,
              shanetworkweights]
    W1s = jnp.stack([pfc_W1, mdcbc_W1, v1c_W1, sha_W1])
    b1s = jnp.stack([pfc_b1, mdcbc_b1, v1c_b1, sha_b1])
    gs = jnp.stack([pfc_g, mdcbc_g, v1c_g, sha_g])
    bts = jnp.stack([pfc_bt, mdcbc_bt, v1c_bt, sha_bt])
    W2s = jnp.stack([pfc_W2, mdcbc_W2, v1c_W2, sha_W2])
    b2s = jnp.stack([pfc_b2, mdcbc_b2, v1c_b2, sha_b2])

    srcs, dsts, ewws = [], [], []
    for ei, ew in zip(eis, ews_in):
        s, d, w = _pad_edges(ei, ew)
        srcs.append(s)
        dsts.append(d)
        ewws.append(w)

    degp = _deg_pass(*dsts, *ewws).reshape(NC, 4, N_PAD)
    deg = degp[0] + degp[1] + 2.0
    dinv = jnp.where(deg > 0, lax.rsqrt(jnp.where(deg > 0, deg, 1.0)), 0.0)

    flat_p = jnp.pad(flatten, ((0, N_PAD - N), (0, 0)))
    h1 = jnp.einsum("nf,gfh->gnh", flat_p, W1s)
    yy1 = dinv[:, :, None] * h1

    zp1 = _prop_pass(yy1[0], yy1[1], yy1[2], yy1[3], *srcs, *dsts,
                     *ewws).reshape(NC, 4, N_PAD, H)
    z1 = zp1[0] + zp1[1]
    out1 = dinv[:, :, None] * (z1 + 2.0 * yy1) + b1s[:, None, :]
    a = jax.nn.relu(out1)
    mu = jnp.mean(a[:, :N, :], axis=1, keepdims=True)
    var = jnp.mean((a[:, :N, :] - mu) ** 2, axis=1, keepdims=True)
    h2 = (a - mu) / jnp.sqrt(var + 1e-5) * gs[:, None, :] + bts[:, None, :]
    yy2 = dinv[:, :, None] * h2

    zp2 = _prop_pass(yy2[0], yy2[1], yy2[2], yy2[3], *srcs, *dsts,
                     *ewws).reshape(NC, 4, N_PAD, H)
    z2 = zp2[0] + zp2[1]
    pre = dinv[:, :, None] * (z2 + 2.0 * yy2)
    logits = jnp.einsum("gnh,ghk->gnk", pre, W2s) + b2s[:, None, :]
    experts = jax.nn.log_softmax(logits, axis=-1)

    w = jax.nn.softmax(features @ gate_W + gate_b, axis=1)
    out = jnp.einsum("ng,gnk->nk", w, experts[:, :N, :])
    return out


# speedup vs baseline: 29.9257x; 1.0829x over previous
import functools

import jax
import jax.numpy as jnp
from jax import lax
from jax.experimental import pallas as pl
from jax.experimental.pallas import tpu as pltpu
from jax.experimental.pallas import tpu_sc as plsc

N = 50000
E = 1600000
H = 16
NC = 2
NS = 16
NW = NC * NS
N_PAD = 51200
NPT = N_PAD // NS
RW = 400
E_PAD = NW * RW * 128
CHR = 40
NCHUNK = RW // CHR
CHP = 8
NCHUNKP = RW // CHP
ZROW = 400

_mesh = plsc.VectorSubcoreMesh(core_axis_name="c", subcore_axis_name="s")


def _zero_fill(zbuf, rows, width):
    @pl.loop(0, rows, unroll=8)
    def _(i):
        zbuf[i, :] = jnp.zeros((width,), jnp.float32)


@functools.partial(
    pl.kernel,
    out_type=jax.ShapeDtypeStruct((NC * 4 * N_PAD,), jnp.float32),
    mesh=_mesh,
    compiler_params=pltpu.CompilerParams(use_tc_tiling_on_sc=False),
    scratch_types=[
        pltpu.VMEM((CHR, 128), jnp.int32),
        pltpu.VMEM((CHR, 128), jnp.float32),
        pltpu.VMEM((NPT,), jnp.float32),
        pltpu.VMEM_SHARED((N_PAD,), jnp.float32),
        pltpu.VMEM_SHARED((N_PAD,), jnp.float32),
        pltpu.VMEM_SHARED((N_PAD,), jnp.float32),
        pltpu.VMEM_SHARED((N_PAD,), jnp.float32),
        pltpu.SemaphoreType.DMA,
    ],
)
def _deg_pass(dst0, dst1, dst2, dst3, ew0, ew1, ew2, ew3, out,
              dstv, ewv, zbuf, acc0, acc1, acc2, acc3, sem):
    cid = lax.axis_index("c")
    sid = lax.axis_index("s")
    wid = sid * NC + cid
    dsts = [dst0, dst1, dst2, dst3]
    ews = [ew0, ew1, ew2, ew3]
    accs = [acc0, acc1, acc2, acc3]
    @pl.loop(0, NPT // 16, unroll=8)
    def _(i):
        zbuf[pl.ds(i * 16, 16)] = jnp.zeros((16,), jnp.float32)

    for g in range(4):
        pltpu.sync_copy(zbuf, accs[g].at[pl.ds(sid * NPT, NPT)])
    plsc.subcore_barrier()
    for g in range(4):
        @pl.loop(0, NCHUNK)
        def _(c):
            r0 = pl.multiple_of(wid * RW + c * CHR, 8)
            pltpu.sync_copy(dsts[g].at[pl.ds(r0, CHR), :], dstv)
            pltpu.sync_copy(ews[g].at[pl.ds(r0, CHR), :], ewv)
            descs = [
                pltpu.async_copy(ewv.at[j], accs[g].at[dstv.at[j]], sem,
                                 add=True)
                for j in range(CHR)
            ]
            for d in descs:
                d.wait()
    plsc.subcore_barrier()
    for g in range(4):
        off = pl.multiple_of((cid * 4 + g) * N_PAD + sid * NPT, 128)
        pltpu.sync_copy(accs[g].at[pl.ds(sid * NPT, NPT)],
                        out.at[pl.ds(off, NPT)])


@functools.partial(
    pl.kernel,
    out_type=jax.ShapeDtypeStruct((NC * 4 * N_PAD, H), jnp.float32),
    mesh=_mesh,
    compiler_params=pltpu.CompilerParams(use_tc_tiling_on_sc=False),
    scratch_types=[
        pltpu.VMEM((CHP, 128), jnp.int32),
        pltpu.VMEM((CHP, 128), jnp.int32),
        pltpu.VMEM((CHP, 128), jnp.float32),
        pltpu.VMEM((CHP, 128, H), jnp.float32),
        pltpu.VMEM((CHP, 128), jnp.int32),
        pltpu.VMEM((CHP, 128), jnp.int32),
        pltpu.VMEM((CHP, 128), jnp.float32),
        pltpu.VMEM((CHP, 128, H), jnp.float32),
        pltpu.VMEM((ZROW, H), jnp.float32),
        pltpu.VMEM_SHARED((N_PAD, H), jnp.float32),
        pltpu.SemaphoreType.DMA,
        pltpu.SemaphoreType.DMA,
        pltpu.SemaphoreType.DMA,
        pltpu.SemaphoreType.DMA,
        pltpu.SemaphoreType.DMA,
        pltpu.SemaphoreType.DMA,
    ],
)
def _prop_pass(y0, y1, y2, y3,
               src0, src1, src2, src3,
               dst0, dst1, dst2, dst3,
               ew0, ew1, ew2, ew3, out,
               srcv0, dstv0, ewv0, rows0,
               srcv1, dstv1, ewv1, rows1,
               zbuf, acc,
               semL0, semL1, semG0, semG1, semS0, semS1):
    cid = lax.axis_index("c")
    sid = lax.axis_index("s")
    wid = sid * NC + cid
    ys = [y0, y1, y2, y3]
    srcs = [src0, src1, src2, src3]
    dsts = [dst0, dst1, dst2, dst3]
    ews = [ew0, ew1, ew2, ew3]
    srcv = [srcv0, srcv1]
    dstv = [dstv0, dstv1]
    ewv = [ewv0, ewv1]
    rows = [rows0, rows1]
    semL = [semL0, semL1]
    semG = [semG0, semG1]
    semS = [semS0, semS1]
    _zero_fill(zbuf, ZROW, H)

    def zero_acc():
        for k in range(NPT // ZROW):
            pltpu.sync_copy(zbuf, acc.at[pl.ds(sid * NPT + k * ZROW, ZROW), :])

    zero_acc()
    plsc.subcore_barrier()

    for g in range(4):
        def fire_lin(c, b):
            r0 = pl.multiple_of(wid * RW + c * CHP, 8)
            pltpu.async_copy(srcs[g].at[pl.ds(r0, CHP), :], srcv[b], semL[b])
            pltpu.async_copy(dsts[g].at[pl.ds(r0, CHP), :], dstv[b], semL[b])
            pltpu.async_copy(ews[g].at[pl.ds(r0, CHP), :], ewv[b], semL[b])

        def wait_lin(b):
            pltpu.make_async_copy(srcs[g].at[pl.ds(0, CHP), :], srcv[b],
                                  semL[b]).wait()
            pltpu.make_async_copy(dsts[g].at[pl.ds(0, CHP), :], dstv[b],
                                  semL[b]).wait()
            pltpu.make_async_copy(ews[g].at[pl.ds(0, CHP), :], ewv[b],
                                  semL[b]).wait()

        def fire_gather(b):
            for j in range(CHP):
                pltpu.async_copy(ys[g].at[srcv[b].at[j]], rows[b].at[j],
                                 semG[b])

        def wait_gather(b):
            for j in range(CHP):
                pltpu.make_async_copy(ys[g].at[srcv[b].at[j]], rows[b].at[j],
                                      semG[b]).wait()

        def fire_scatter(b):
            for j in range(CHP):
                pltpu.async_copy(rows[b].at[j], acc.at[dstv[b].at[j]],
                                 semS[b], add=True)

        def wait_scatter(b):
            for j in range(CHP):
                pltpu.make_async_copy(rows[b].at[j], acc.at[dstv[b].at[j]],
                                      semS[b]).wait()

        def scale(b):
            @pl.loop(0, CHP)
            def _(r):
                for l0 in range(0, 128, 16):
                    ew16 = ewv[b][r, pl.ds(l0, 16)]
                    for k in range(16):
                        rows[b][r, l0 + k, :] = rows[b][r, l0 + k, :] * ew16[k]

        fire_lin(0, 0)
        fire_lin(1, 1)
        wait_lin(0)
        fire_gather(0)

        @pl.loop(0, NCHUNKP // 2)
        def _(p):
            c0 = 2 * p
            wait_gather(0)
            scale(0)
            fire_scatter(0)
            wait_lin(1)
            fire_gather(1)
            wait_scatter(0)

            @pl.when(p < NCHUNKP // 2 - 1)
            def _():
                fire_lin(c0 + 2, 0)
            wait_gather(1)
            scale(1)
            fire_scatter(1)
            wait_scatter(1)

            @pl.when(p < NCHUNKP // 2 - 1)
            def _():
                fire_lin(c0 + 3, 1)
                wait_lin(0)
                fire_gather(0)

        plsc.subcore_barrier()
        roff = pl.multiple_of((cid * 4 + g) * N_PAD + sid * NPT, 128)
        pltpu.sync_copy(acc.at[pl.ds(sid * NPT, NPT), :],
                        out.at[pl.ds(roff, NPT), :])
        if g < 3:
            zero_acc()
            plsc.subcore_barrier()


def _pad_edges(ei, ew):
    src = jnp.pad(ei[0], (0, E_PAD - E)).reshape(E_PAD // 128, 128)
    dst = jnp.pad(ei[1], (0, E_PAD - E)).reshape(E_PAD // 128, 128)
    eww = jnp.pad(ew, (0, E_PAD - E)).reshape(E_PAD // 128, 128)
    return src, dst, eww


def kernel(flatten, features, pfcnetworks, mdcbcnetworks, v1cnetworks,
           shanetworks, pfcnetworkweights, mdcbcnetworkweights,
           v1cnetworkweights, shanetworkweights,
           pfc_W1, pfc_b1, pfc_g, pfc_bt, pfc_W2, pfc_b2,
           mdcbc_W1, mdcbc_b1, mdcbc_g, mdcbc_bt, mdcbc_W2, mdcbc_b2,
           v1c_W1, v1c_b1, v1c_g, v1c_bt, v1c_W2, v1c_b2,
           sha_W1, sha_b1, sha_g, sha_bt, sha_W2, sha_b2,
           gate_W, gate_b):
    eis = [pfcnetworks, mdcbcnetworks, v1cnetworks, shanetworks]
    ews_in = [pfcnetworkweights, mdcbcnetworkweights, v1cnetworkweights,
              shanetworkweights]
    W1s = jnp.stack([pfc_W1, mdcbc_W1, v1c_W1, sha_W1])
    b1s = jnp.stack([pfc_b1, mdcbc_b1, v1c_b1, sha_b1])
    gs = jnp.stack([pfc_g, mdcbc_g, v1c_g, sha_g])
    bts = jnp.stack([pfc_bt, mdcbc_bt, v1c_bt, sha_bt])
    W2s = jnp.stack([pfc_W2, mdcbc_W2, v1c_W2, sha_W2])
    b2s = jnp.stack([pfc_b2, mdcbc_b2, v1c_b2, sha_b2])

    srcs, dsts, ewws = [], [], []
    for ei, ew in zip(eis, ews_in):
        s, d, w = _pad_edges(ei, ew)
        srcs.append(s)
        dsts.append(d)
        ewws.append(w)

    degp = _deg_pass(*dsts, *ewws).reshape(NC, 4, N_PAD)
    deg = degp[0] + degp[1] + 2.0
    dinv = jnp.where(deg > 0, lax.rsqrt(jnp.where(deg > 0, deg, 1.0)), 0.0)

    flat_p = jnp.pad(flatten, ((0, N_PAD - N), (0, 0)))
    h1 = jnp.einsum("nf,gfh->gnh", flat_p, W1s)
    yy1 = dinv[:, :, None] * h1

    zp1 = _prop_pass(yy1[0], yy1[1], yy1[2], yy1[3], *srcs, *dsts,
                     *ewws).reshape(NC, 4, N_PAD, H)
    z1 = zp1[0] + zp1[1]
    out1 = dinv[:, :, None] * (z1 + 2.0 * yy1) + b1s[:, None, :]
    a = jax.nn.relu(out1)
    mu = jnp.mean(a[:, :N, :], axis=1, keepdims=True)
    var = jnp.mean((a[:, :N, :] - mu) ** 2, axis=1, keepdims=True)
    h2 = (a - mu) / jnp.sqrt(var + 1e-5) * gs[:, None, :] + bts[:, None, :]
    yy2 = dinv[:, :, None] * h2

    zp2 = _prop_pass(yy2[0], yy2[1], yy2[2], yy2[3], *srcs, *dsts,
                     *ewws).reshape(NC, 4, N_PAD, H)
    z2 = zp2[0] + zp2[1]
    pre = dinv[:, :, None] * (z2 + 2.0 * yy2)
    logits = jnp.einsum("gnh,ghk->gnk", pre, W2s) + b2s[:, None, :]
    experts = jax.nn.log_softmax(logits, axis=-1)

    w = jax.nn.softmax(features @ gate_W + gate_b, axis=1)
    out = jnp.einsum("ng,gnk->nk", w, experts[:, :N, :])
    return out


# speedup vs baseline: 42.0104x; 1.4038x over previous
import functools

import jax
import jax.numpy as jnp
from jax import lax
from jax.experimental import pallas as pl
from jax.experimental.pallas import tpu as pltpu
from jax.experimental.pallas import tpu_sc as plsc

N = 50000
E = 1600000
H = 16
NC = 2
NS = 16
NW = NC * NS
N_PAD = 51200
NPT = N_PAD // NS
RT = E // 128
PAIR_ROWS = 784
L0 = 232
L1 = PAIR_ROWS - L0
CHP = 8
ZROW = 400
LASTLEN = RT - (NS - 1) * PAIR_ROWS - L0
PART0 = RT - CHP
PARTJLO = CHP - (LASTLEN - (LASTLEN // CHP) * CHP)

_mesh = plsc.VectorSubcoreMesh(core_axis_name="c", subcore_axis_name="s")


def _zero_fill(zbuf, rows, width):
    @pl.loop(0, rows, unroll=8)
    def _(i):
        zbuf[i, :] = jnp.zeros((width,), jnp.float32)


def _worker_plan(cid, sid):
    base = sid * PAIR_ROWS + cid * L0
    nch = jnp.where(
        cid == 0, L0 // CHP,
        jnp.where(sid == NS - 1, LASTLEN // CHP, L1 // CHP))
    return base, nch


@functools.partial(
    pl.kernel,
    out_type=jax.ShapeDtypeStruct((NC * 4 * N_PAD,), jnp.float32),
    mesh=_mesh,
    compiler_params=pltpu.CompilerParams(use_tc_tiling_on_sc=False),
    scratch_types=[
        pltpu.VMEM((CHP, 128), jnp.int32),
        pltpu.VMEM((CHP, 128), jnp.float32),
        pltpu.VMEM((NPT,), jnp.float32),
        pltpu.VMEM_SHARED((N_PAD,), jnp.float32),
        pltpu.VMEM_SHARED((N_PAD,), jnp.float32),
        pltpu.VMEM_SHARED((N_PAD,), jnp.float32),
        pltpu.VMEM_SHARED((N_PAD,), jnp.float32),
        pltpu.SemaphoreType.DMA,
    ],
)
def _deg_pass(e0, e1, e2, e3, w0, w1, w2, w3, out,
              dstv, ewv, zbuf, acc0, acc1, acc2, acc3, sem):
    cid = lax.axis_index("c")
    sid = lax.axis_index("s")
    eis = [e0, e1, e2, e3]
    ews = [w0, w1, w2, w3]
    accs = [acc0, acc1, acc2, acc3]
    base, nch = _worker_plan(cid, sid)

    @pl.loop(0, NPT // 16, unroll=8)
    def _(i):
        zbuf[pl.ds(i * 16, 16)] = jnp.zeros((16,), jnp.float32)

    for g in range(4):
        pltpu.sync_copy(zbuf, accs[g].at[pl.ds(sid * NPT, NPT)])
    plsc.subcore_barrier()

    for g in range(4):
        def chunk(r0, jlo):
            pltpu.sync_copy(eis[g].at[1, pl.ds(r0, CHP), :], dstv)
            pltpu.sync_copy(ews[g].at[pl.ds(r0, CHP), :], ewv)
            ds_ = [pltpu.async_copy(ewv.at[j], accs[g].at[dstv.at[j]], sem,
                                    add=True)
                   for j in range(jlo, CHP)]
            for d in ds_:
                d.wait()

        @pl.loop(0, nch)
        def _(c):
            chunk(pl.multiple_of(base + c * CHP, 8), 0)

        @pl.when(jnp.logical_and(cid == 1, sid == NS - 1))
        def _():
            chunk(PART0, PARTJLO)

    plsc.subcore_barrier()
    for g in range(4):
        off = pl.multiple_of((cid * 4 + g) * N_PAD + sid * NPT, 128)
        pltpu.sync_copy(accs[g].at[pl.ds(sid * NPT, NPT)],
                        out.at[pl.ds(off, NPT)])


@functools.partial(
    pl.kernel,
    out_type=jax.ShapeDtypeStruct((NC * 4 * N_PAD, H), jnp.float32),
    mesh=_mesh,
    compiler_params=pltpu.CompilerParams(use_tc_tiling_on_sc=False),
    scratch_types=[
        pltpu.VMEM((CHP, 128), jnp.int32),
        pltpu.VMEM((CHP, 128), jnp.int32),
        pltpu.VMEM((CHP, 128), jnp.float32),
        pltpu.VMEM((CHP, 128, H), jnp.float32),
        pltpu.VMEM((CHP, 128), jnp.int32),
        pltpu.VMEM((CHP, 128), jnp.int32),
        pltpu.VMEM((CHP, 128), jnp.float32),
        pltpu.VMEM((CHP, 128, H), jnp.float32),
        pltpu.VMEM((ZROW, H), jnp.float32),
        pltpu.VMEM_SHARED((N_PAD, H), jnp.float32),
        pltpu.SemaphoreType.DMA,
        pltpu.SemaphoreType.DMA,
        pltpu.SemaphoreType.DMA,
        pltpu.SemaphoreType.DMA,
        pltpu.SemaphoreType.DMA,
        pltpu.SemaphoreType.DMA,
    ],
)
def _prop_pass(y0, y1, y2, y3,
               e0, e1, e2, e3,
               w0, w1, w2, w3, out,
               srcv0, dstv0, ewv0, rows0,
               srcv1, dstv1, ewv1, rows1,
               zbuf, acc,
               semL0, semL1, semG0, semG1, semS0, semS1):
    cid = lax.axis_index("c")
    sid = lax.axis_index("s")
    ys = [y0, y1, y2, y3]
    eis = [e0, e1, e2, e3]
    ews = [w0, w1, w2, w3]
    srcv = [srcv0, srcv1]
    dstv = [dstv0, dstv1]
    ewv = [ewv0, ewv1]
    rows = [rows0, rows1]
    semL = [semL0, semL1]
    semG = [semG0, semG1]
    semS = [semS0, semS1]
    base, nch = _worker_plan(cid, sid)
    npair = (nch - 1) // 2
    _zero_fill(zbuf, ZROW, H)

    def zero_acc():
        for k in range(NPT // ZROW):
            pltpu.sync_copy(zbuf, acc.at[pl.ds(sid * NPT + k * ZROW, ZROW), :])

    zero_acc()
    plsc.subcore_barrier()

    for g in range(4):
        def fire_lin(c, b):
            r0 = pl.multiple_of(base + c * CHP, 8)
            pltpu.async_copy(eis[g].at[0, pl.ds(r0, CHP), :], srcv[b], semL[b])
            pltpu.async_copy(eis[g].at[1, pl.ds(r0, CHP), :], dstv[b], semL[b])
            pltpu.async_copy(ews[g].at[pl.ds(r0, CHP), :], ewv[b], semL[b])

        def wait_lin(b):
            pltpu.make_async_copy(eis[g].at[0, pl.ds(0, CHP), :], srcv[b],
                                  semL[b]).wait()
            pltpu.make_async_copy(eis[g].at[1, pl.ds(0, CHP), :], dstv[b],
                                  semL[b]).wait()
            pltpu.make_async_copy(ews[g].at[pl.ds(0, CHP), :], ewv[b],
                                  semL[b]).wait()

        def fire_gather(b, jlo=0):
            for j in range(jlo, CHP):
                pltpu.async_copy(ys[g].at[srcv[b].at[j]], rows[b].at[j],
                                 semG[b])

        def wait_gather(b, jlo=0):
            for j in range(jlo, CHP):
                pltpu.make_async_copy(ys[g].at[srcv[b].at[j]], rows[b].at[j],
                                      semG[b]).wait()

        def fire_scatter(b, jlo=0):
            for j in range(jlo, CHP):
                pltpu.async_copy(rows[b].at[j], acc.at[dstv[b].at[j]],
                                 semS[b], add=True)

        def wait_scatter(b, jlo=0):
            for j in range(jlo, CHP):
                pltpu.make_async_copy(rows[b].at[j], acc.at[dstv[b].at[j]],
                                      semS[b]).wait()

        def scale(b):
            @pl.loop(0, CHP)
            def _(r):
                for l0 in range(0, 128, 16):
                    ew16 = ewv[b][r, pl.ds(l0, 16)]
                    for k in range(16):
                        rows[b][r, l0 + k, :] = rows[b][r, l0 + k, :] * ew16[k]

        fire_lin(0, 0)
        fire_lin(1, 1)
        wait_lin(0)
        fire_gather(0)

        @pl.loop(0, npair)
        def _(p):
            c0 = 2 * p
            wait_gather(0)
            scale(0)
            fire_scatter(0)
            wait_lin(1)
            fire_gather(1)
            wait_scatter(0)

            @pl.when(p < npair - 1)
            def _():
                fire_lin(c0 + 2, 0)
            wait_gather(1)
            scale(1)
            fire_scatter(1)
            wait_scatter(1)

            @pl.when(p < npair - 1)
            def _():
                fire_lin(c0 + 3, 1)
                wait_lin(0)
                fire_gather(0)

        fire_lin(nch - 1, 0)
        wait_lin(0)
        fire_gather(0)
        wait_gather(0)
        scale(0)
        fire_scatter(0)
        wait_scatter(0)

        @pl.when(jnp.logical_and(cid == 1, sid == NS - 1))
        def _():
            pltpu.sync_copy(eis[g].at[0, pl.ds(PART0, CHP), :], srcv[0])
            pltpu.sync_copy(eis[g].at[1, pl.ds(PART0, CHP), :], dstv[0])
            pltpu.sync_copy(ews[g].at[pl.ds(PART0, CHP), :], ewv[0])
            fire_gather(0, PARTJLO)
            wait_gather(0, PARTJLO)
            scale(0)
            fire_scatter(0, PARTJLO)
            wait_scatter(0, PARTJLO)

        plsc.subcore_barrier()
        roff = pl.multiple_of((cid * 4 + g) * N_PAD + sid * NPT, 128)
        pltpu.sync_copy(acc.at[pl.ds(sid * NPT, NPT), :],
                        out.at[pl.ds(roff, NPT), :])
        if g < 3:
            zero_acc()
            plsc.subcore_barrier()


def kernel(flatten, features, pfcnetworks, mdcbcnetworks, v1cnetworks,
           shanetworks, pfcnetworkweights, mdcbcnetworkweights,
           v1cnetworkweights, shanetworkweights,
           pfc_W1, pfc_b1, pfc_g, pfc_bt, pfc_W2, pfc_b2,
           mdcbc_W1, mdcbc_b1, mdcbc_g, mdcbc_bt, mdcbc_W2, mdcbc_b2,
           v1c_W1, v1c_b1, v1c_g, v1c_bt, v1c_W2, v1c_b2,
           sha_W1, sha_b1, sha_g, sha_bt, sha_W2, sha_b2,
           gate_W, gate_b):
    eis = [pfcnetworks, mdcbcnetworks, v1cnetworks, shanetworks]
    ews_in = [pfcnetworkweights, mdcbcnetworkweights, v1cnetworkweights,
              shanetworkweights]
    W1s = jnp.stack([pfc_W1, mdcbc_W1, v1c_W1, sha_W1])
    b1s = jnp.stack([pfc_b1, mdcbc_b1, v1c_b1, sha_b1])
    gs = jnp.stack([pfc_g, mdcbc_g, v1c_g, sha_g])
    bts = jnp.stack([pfc_bt, mdcbc_bt, v1c_bt, sha_bt])
    W2s = jnp.stack([pfc_W2, mdcbc_W2, v1c_W2, sha_W2])
    b2s = jnp.stack([pfc_b2, mdcbc_b2, v1c_b2, sha_b2])

    e3 = [ei.reshape(2, RT, 128) for ei in eis]
    w2d = [ew.reshape(RT, 128) for ew in ews_in]

    degp = _deg_pass(*e3, *w2d).reshape(NC, 4, N_PAD)
    deg = degp[0] + degp[1] + 2.0
    dinv = jnp.where(deg > 0, lax.rsqrt(jnp.where(deg > 0, deg, 1.0)), 0.0)

    flat_p = jnp.pad(flatten, ((0, N_PAD - N), (0, 0)))
    h1 = jnp.einsum("nf,gfh->gnh", flat_p, W1s)
    yy1 = dinv[:, :, None] * h1

    zp1 = _prop_pass(yy1[0], yy1[1], yy1[2], yy1[3], *e3,
                     *w2d).reshape(NC, 4, N_PAD, H)
    z1 = zp1[0] + zp1[1]
    out1 = dinv[:, :, None] * (z1 + 2.0 * yy1) + b1s[:, None, :]
    a = jax.nn.relu(out1)
    mu = jnp.mean(a[:, :N, :], axis=1, keepdims=True)
    var = jnp.mean((a[:, :N, :] - mu) ** 2, axis=1, keepdims=True)
    h2 = (a - mu) / jnp.sqrt(var + 1e-5) * gs[:, None, :] + bts[:, None, :]
    yy2 = dinv[:, :, None] * h2

    zp2 = _prop_pass(yy2[0], yy2[1], yy2[2], yy2[3], *e3,
                     *w2d).reshape(NC, 4, N_PAD, H)
    z2 = zp2[0] + zp2[1]
    pre = dinv[:, :, None] * (z2 + 2.0 * yy2)
    logits = jnp.einsum("gnh,ghk->gnk", pre, W2s) + b2s[:, None, :]
    experts = jax.nn.log_softmax(logits, axis=-1)

    w = jax.nn.softmax(features @ gate_W + gate_b, axis=1)
    out = jnp.einsum("ng,gnk->nk", w, experts[:, :N, :])
    return out


# speedup vs baseline: 56.1525x; 1.3366x over previous
import functools

import jax
import jax.numpy as jnp
from jax import lax
from jax.experimental import pallas as pl
from jax.experimental.pallas import tpu as pltpu
from jax.experimental.pallas import tpu_sc as plsc

N = 50000
E = 1600000
H = 16
NC = 2
NS = 16
NW = NC * NS
N_PAD = 51200
NPT = N_PAD // NS
RT = E // 128
PAIR_ROWS = 784
L0 = 392
L1 = PAIR_ROWS - L0
CHP = 8
ZROW = 400
LASTLEN = RT - (NS - 1) * PAIR_ROWS - L0
PART0 = RT - CHP
PARTJLO = CHP - (LASTLEN - (LASTLEN // CHP) * CHP)

_mesh = plsc.VectorSubcoreMesh(core_axis_name="c", subcore_axis_name="s")


def _zero_fill(zbuf, rows, width):
    @pl.loop(0, rows, unroll=8)
    def _(i):
        zbuf[i, :] = jnp.zeros((width,), jnp.float32)


def _worker_plan(cid, sid):
    base = sid * PAIR_ROWS + cid * L0
    nch = jnp.where(
        cid == 0, L0 // CHP,
        jnp.where(sid == NS - 1, LASTLEN // CHP, L1 // CHP))
    return base, nch


@functools.partial(
    pl.kernel,
    out_type=jax.ShapeDtypeStruct((NC * 4 * N_PAD,), jnp.float32),
    mesh=_mesh,
    compiler_params=pltpu.CompilerParams(use_tc_tiling_on_sc=False),
    scratch_types=[
        pltpu.VMEM((CHP, 128), jnp.int32),
        pltpu.VMEM((CHP, 128), jnp.float32),
        pltpu.VMEM((NPT,), jnp.float32),
        pltpu.VMEM_SHARED((N_PAD,), jnp.float32),
        pltpu.VMEM_SHARED((N_PAD,), jnp.float32),
        pltpu.VMEM_SHARED((N_PAD,), jnp.float32),
        pltpu.VMEM_SHARED((N_PAD,), jnp.float32),
        pltpu.SemaphoreType.DMA,
    ],
)
def _deg_pass(e0, e1, e2, e3, w0, w1, w2, w3, out,
              dstv, ewv, zbuf, acc0, acc1, acc2, acc3, sem):
    cid = lax.axis_index("c")
    sid = lax.axis_index("s")
    eis = [e0, e1, e2, e3]
    ews = [w0, w1, w2, w3]
    accs = [acc0, acc1, acc2, acc3]
    base, nch = _worker_plan(cid, sid)

    @pl.loop(0, NPT // 16, unroll=8)
    def _(i):
        zbuf[pl.ds(i * 16, 16)] = jnp.zeros((16,), jnp.float32)

    for g in range(4):
        pltpu.sync_copy(zbuf, accs[g].at[pl.ds(sid * NPT, NPT)])
    plsc.subcore_barrier()

    for g in range(4):
        def chunk(r0, jlo):
            pltpu.sync_copy(eis[g].at[1, pl.ds(r0, CHP), :], dstv)
            pltpu.sync_copy(ews[g].at[pl.ds(r0, CHP), :], ewv)
            ds_ = [pltpu.async_copy(ewv.at[j], accs[g].at[dstv.at[j]], sem,
                                    add=True)
                   for j in range(jlo, CHP)]
            for d in ds_:
                d.wait()

        @pl.loop(0, nch)
        def _(c):
            chunk(pl.multiple_of(base + c * CHP, 8), 0)

        @pl.when(jnp.logical_and(cid == 1, sid == NS - 1))
        def _():
            chunk(PART0, PARTJLO)

    plsc.subcore_barrier()
    for g in range(4):
        off = pl.multiple_of((cid * 4 + g) * N_PAD + sid * NPT, 128)
        pltpu.sync_copy(accs[g].at[pl.ds(sid * NPT, NPT)],
                        out.at[pl.ds(off, NPT)])


@functools.partial(
    pl.kernel,
    out_type=jax.ShapeDtypeStruct((NC * 4 * N_PAD, H), jnp.float32),
    mesh=_mesh,
    compiler_params=pltpu.CompilerParams(use_tc_tiling_on_sc=False),
    scratch_types=[
        pltpu.VMEM((CHP, 128), jnp.int32),
        pltpu.VMEM((CHP, 128), jnp.int32),
        pltpu.VMEM((CHP, 128), jnp.float32),
        pltpu.VMEM((CHP, 128, H), jnp.float32),
        pltpu.VMEM((CHP, 128), jnp.int32),
        pltpu.VMEM((CHP, 128), jnp.int32),
        pltpu.VMEM((CHP, 128), jnp.float32),
        pltpu.VMEM((CHP, 128, H), jnp.float32),
        pltpu.VMEM((ZROW, H), jnp.float32),
        pltpu.VMEM_SHARED((N_PAD, H), jnp.float32),
        pltpu.SemaphoreType.DMA,
        pltpu.SemaphoreType.DMA,
        pltpu.SemaphoreType.DMA,
        pltpu.SemaphoreType.DMA,
        pltpu.SemaphoreType.DMA,
        pltpu.SemaphoreType.DMA,
    ],
)
def _prop_pass(y0, y1, y2, y3,
               e0, e1, e2, e3,
               w0, w1, w2, w3, out,
               srcv0, dstv0, ewv0, rows0,
               srcv1, dstv1, ewv1, rows1,
               zbuf, acc,
               semL0, semL1, semG0, semG1, semS0, semS1):
    cid = lax.axis_index("c")
    sid = lax.axis_index("s")
    ys = [y0, y1, y2, y3]
    eis = [e0, e1, e2, e3]
    ews = [w0, w1, w2, w3]
    srcv = [srcv0, srcv1]
    dstv = [dstv0, dstv1]
    ewv = [ewv0, ewv1]
    rows = [rows0, rows1]
    semL = [semL0, semL1]
    semG = [semG0, semG1]
    semS = [semS0, semS1]
    base, nch = _worker_plan(cid, sid)
    npair = (nch - 1) // 2
    _zero_fill(zbuf, ZROW, H)

    def zero_acc():
        for k in range(NPT // ZROW):
            pltpu.sync_copy(zbuf, acc.at[pl.ds(sid * NPT + k * ZROW, ZROW), :])

    zero_acc()
    plsc.subcore_barrier()

    for g in range(4):
        def fire_lin(c, b):
            r0 = pl.multiple_of(base + c * CHP, 8)
            pltpu.async_copy(eis[g].at[0, pl.ds(r0, CHP), :], srcv[b], semL[b])
            pltpu.async_copy(eis[g].at[1, pl.ds(r0, CHP), :], dstv[b], semL[b])
            pltpu.async_copy(ews[g].at[pl.ds(r0, CHP), :], ewv[b], semL[b])

        def wait_lin(b):
            pltpu.make_async_copy(eis[g].at[0, pl.ds(0, CHP), :], srcv[b],
                                  semL[b]).wait()
            pltpu.make_async_copy(eis[g].at[1, pl.ds(0, CHP), :], dstv[b],
                                  semL[b]).wait()
            pltpu.make_async_copy(ews[g].at[pl.ds(0, CHP), :], ewv[b],
                                  semL[b]).wait()

        def fire_gather(b, jlo=0):
            for j in range(jlo, CHP):
                pltpu.async_copy(ys[g].at[srcv[b].at[j]], rows[b].at[j],
                                 semG[b])

        def wait_gather(b, jlo=0):
            for j in range(jlo, CHP):
                pltpu.make_async_copy(ys[g].at[srcv[b].at[j]], rows[b].at[j],
                                      semG[b]).wait()

        def fire_scatter(b, jlo=0):
            for j in range(jlo, CHP):
                pltpu.async_copy(rows[b].at[j], acc.at[dstv[b].at[j]],
                                 semS[b], add=True)

        def wait_scatter(b, jlo=0):
            for j in range(jlo, CHP):
                pltpu.make_async_copy(rows[b].at[j], acc.at[dstv[b].at[j]],
                                      semS[b]).wait()

        def scale(b):
            @pl.loop(0, CHP)
            def _(r):
                for l0 in range(0, 128, 16):
                    ew16 = ewv[b][r, pl.ds(l0, 16)]
                    for k in range(16):
                        rows[b][r, l0 + k, :] = rows[b][r, l0 + k, :] * ew16[k]

        fire_lin(0, 0)
        fire_lin(1, 1)
        wait_lin(0)
        fire_gather(0)

        @pl.loop(0, npair)
        def _(p):
            c0 = 2 * p
            wait_gather(0)
            scale(0)
            fire_scatter(0)
            wait_lin(1)
            fire_gather(1)
            wait_scatter(0)

            @pl.when(p < npair - 1)
            def _():
                fire_lin(c0 + 2, 0)
            wait_gather(1)
            scale(1)
            fire_scatter(1)
            wait_scatter(1)

            @pl.when(p < npair - 1)
            def _():
                fire_lin(c0 + 3, 1)
                wait_lin(0)
                fire_gather(0)

        fire_lin(nch - 1, 0)
        wait_lin(0)
        fire_gather(0)
        wait_gather(0)
        scale(0)
        fire_scatter(0)
        wait_scatter(0)

        @pl.when(jnp.logical_and(cid == 1, sid == NS - 1))
        def _():
            pltpu.sync_copy(eis[g].at[0, pl.ds(PART0, CHP), :], srcv[0])
            pltpu.sync_copy(eis[g].at[1, pl.ds(PART0, CHP), :], dstv[0])
            pltpu.sync_copy(ews[g].at[pl.ds(PART0, CHP), :], ewv[0])
            fire_gather(0, PARTJLO)
            wait_gather(0, PARTJLO)
            scale(0)
            fire_scatter(0, PARTJLO)
            wait_scatter(0, PARTJLO)

        plsc.subcore_barrier()
        roff = pl.multiple_of((cid * 4 + g) * N_PAD + sid * NPT, 128)
        pltpu.sync_copy(acc.at[pl.ds(sid * NPT, NPT), :],
                        out.at[pl.ds(roff, NPT), :])
        if g < 3:
            zero_acc()
            plsc.subcore_barrier()


X8 = N_PAD // 8
XV = N // 8
DR = N_PAD // 128


def kernel(flatten, features, pfcnetworks, mdcbcnetworks, v1cnetworks,
           shanetworks, pfcnetworkweights, mdcbcnetworkweights,
           v1cnetworkweights, shanetworkweights,
           pfc_W1, pfc_b1, pfc_g, pfc_bt, pfc_W2, pfc_b2,
           mdcbc_W1, mdcbc_b1, mdcbc_g, mdcbc_bt, mdcbc_W2, mdcbc_b2,
           v1c_W1, v1c_b1, v1c_g, v1c_bt, v1c_W2, v1c_b2,
           sha_W1, sha_b1, sha_g, sha_bt, sha_W2, sha_b2,
           gate_W, gate_b):
    eis = [pfcnetworks, mdcbcnetworks, v1cnetworks, shanetworks]
    ews_in = [pfcnetworkweights, mdcbcnetworkweights, v1cnetworkweights,
              shanetworkweights]
    W1s = jnp.stack([pfc_W1, mdcbc_W1, v1c_W1, sha_W1])
    b1s = jnp.stack([pfc_b1, mdcbc_b1, v1c_b1, sha_b1])
    gs = jnp.stack([pfc_g, mdcbc_g, v1c_g, sha_g])
    bts = jnp.stack([pfc_bt, mdcbc_bt, v1c_bt, sha_bt])
    W2s = jnp.stack([pfc_W2, mdcbc_W2, v1c_W2, sha_W2])
    b2s = jnp.stack([pfc_b2, mdcbc_b2, v1c_b2, sha_b2])

    e3 = [ei.reshape(2, RT, 128) for ei in eis]
    w2d = [ew.reshape(RT, 128) for ew in ews_in]

    eye8 = jnp.eye(8, dtype=jnp.float32)
    w1bd = jnp.einsum("ab,gfh->gafbh", eye8, W1s).reshape(4, 120, 128)
    w1bd = jnp.pad(w1bd, ((0, 0), (0, 8), (0, 0)))
    w2p = jnp.pad(W2s, ((0, 0), (0, 0), (0, 14)))
    w2bd = jnp.einsum("ab,ghj->gahbj", eye8, w2p).reshape(4, 128, 128)
    b2t = jnp.tile(jnp.pad(b2s, ((0, 0), (0, 14))), (1, 8))
    b1t = jnp.tile(b1s, (1, 8))
    lane = jnp.arange(128)
    swap = jnp.where(lane % 16 == 0, lane + 1,
                     jnp.where(lane % 16 == 1, lane - 1, lane))
    perm = jax.nn.one_hot(swap, 128, dtype=jnp.float32)

    degp = _deg_pass(*e3, *w2d)
    degv = degp.reshape(NC * 4 * DR, 128)
    deg = (degv[:4 * DR] + degv[4 * DR:]).reshape(4, DR, 128) + 2.0
    dinv = jnp.where(deg > 0, lax.rsqrt(jnp.where(deg > 0, deg, 1.0)), 0.0)
    dinv16 = jnp.repeat(dinv, 16, axis=-1).reshape(4, X8, 128)

    fp = flatten.reshape(XV, 120)
    fp = jnp.pad(fp, ((0, X8 - XV), (0, 8)))
    h1 = jnp.einsum("xl,glo->gxo", fp, w1bd)
    y1 = dinv16 * h1

    zp1 = _prop_pass(*[y1[g].reshape(N_PAD, H) for g in range(4)],
                     *e3, *w2d)
    zv1 = zp1.reshape(NC * 4 * X8, 128)
    z1 = (zv1[:4 * X8] + zv1[4 * X8:]).reshape(4, X8, 128)
    a = jax.nn.relu(dinv16 * (z1 + 2.0 * y1) + b1t[:, None, :])
    s1 = jnp.sum(a[:, :XV], axis=1).reshape(4, 8, 16).sum(axis=1)
    s2 = jnp.sum(a[:, :XV] * a[:, :XV], axis=1).reshape(4, 8, 16).sum(axis=1)
    mu = s1 / N
    var = s2 / N - mu * mu
    alpha = gs * lax.rsqrt(var + 1e-5)
    beta = bts - mu * alpha
    alphat = jnp.tile(alpha, (1, 8))
    betat = jnp.tile(beta, (1, 8))
    y2 = dinv16 * (a * alphat[:, None, :] + betat[:, None, :])

    zp2 = _prop_pass(*[y2[g].reshape(N_PAD, H) for g in range(4)],
                     *e3, *w2d)
    zv2 = zp2.reshape(NC * 4 * X8, 128)
    z2 = (zv2[:4 * X8] + zv2[4 * X8:]).reshape(4, X8, 128)
    pre = dinv16 * (z2 + 2.0 * y2)
    logits = jnp.einsum("gxl,glo->gxo", pre, w2bd) + b2t[:, None, :]
    partner = jnp.einsum("gxl,lo->gxo", logits, perm)
    m = jnp.maximum(logits, partner)
    ls = logits - (m + jnp.log(jnp.exp(logits - m) + jnp.exp(partner - m)))

    glt = gate_W.T @ features.T + gate_b[:, None]
    wt = jax.nn.softmax(glt, axis=0)
    wtp = jnp.pad(wt, ((0, 0), (0, N_PAD - N))).reshape(4, DR, 128)
    wpk = jnp.repeat(wtp, 16, axis=-1).reshape(4, X8, 128)
    mixed = jnp.sum(wpk * ls, axis=0)
    return mixed.reshape(N_PAD, H)[:N, :2]


# speedup vs baseline: 68.1694x; 1.2140x over previous
import functools

import jax
import jax.numpy as jnp
from jax import lax
from jax.experimental import pallas as pl
from jax.experimental.pallas import tpu as pltpu
from jax.experimental.pallas import tpu_sc as plsc

N = 50000
E = 1600000
H = 16
NC = 2
NS = 16
NW = NC * NS
N_PAD = 51200
NPT = N_PAD // NS
RT = E // 128
PAIR_ROWS = 784
L0 = 392
L1 = PAIR_ROWS - L0
CHP = 8
ZROW = 400
LASTLEN = RT - (NS - 1) * PAIR_ROWS - L0
PART0 = RT - CHP
PARTJLO = CHP - (LASTLEN - (LASTLEN // CHP) * CHP)

_mesh = plsc.VectorSubcoreMesh(core_axis_name="c", subcore_axis_name="s")


def _zero_fill(zbuf, rows, width):
    @pl.loop(0, rows, unroll=8)
    def _(i):
        zbuf[i, :] = jnp.zeros((width,), jnp.float32)


def _worker_plan(cid, sid):
    base = sid * PAIR_ROWS + cid * L0
    nch = jnp.where(
        cid == 0, L0 // CHP,
        jnp.where(sid == NS - 1, LASTLEN // CHP, L1 // CHP))
    return base, nch


@functools.partial(
    pl.kernel,
    out_type=jax.ShapeDtypeStruct((NC * 4 * N_PAD,), jnp.float32),
    mesh=_mesh,
    compiler_params=pltpu.CompilerParams(use_tc_tiling_on_sc=False),
    scratch_types=[
        pltpu.VMEM((CHP, 128), jnp.int32),
        pltpu.VMEM((CHP, 128), jnp.float32),
        pltpu.VMEM((CHP, 128), jnp.int32),
        pltpu.VMEM((CHP, 128), jnp.float32),
        pltpu.VMEM((NPT,), jnp.float32),
        pltpu.VMEM_SHARED((N_PAD,), jnp.float32),
        pltpu.VMEM_SHARED((N_PAD,), jnp.float32),
        pltpu.VMEM_SHARED((N_PAD,), jnp.float32),
        pltpu.VMEM_SHARED((N_PAD,), jnp.float32),
        pltpu.SemaphoreType.DMA,
        pltpu.SemaphoreType.DMA,
        pltpu.SemaphoreType.DMA,
        pltpu.SemaphoreType.DMA,
    ],
)
def _deg_pass(e0, e1, e2, e3, w0, w1, w2, w3, out,
              dstv0, ewv0, dstv1, ewv1, zbuf, acc0, acc1, acc2, acc3,
              semL0, semL1, semS0, semS1):
    cid = lax.axis_index("c")
    sid = lax.axis_index("s")
    eis = [e0, e1, e2, e3]
    ews = [w0, w1, w2, w3]
    accs = [acc0, acc1, acc2, acc3]
    dstv = [dstv0, dstv1]
    ewv = [ewv0, ewv1]
    semL = [semL0, semL1]
    semS = [semS0, semS1]
    base, nch = _worker_plan(cid, sid)
    npair = (nch - 1) // 2

    @pl.loop(0, NPT // 16, unroll=8)
    def _(i):
        zbuf[pl.ds(i * 16, 16)] = jnp.zeros((16,), jnp.float32)

    for g in range(4):
        pltpu.sync_copy(zbuf, accs[g].at[pl.ds(sid * NPT, NPT)])
    plsc.subcore_barrier()

    for g in range(4):
        def fire_ld(c, b):
            r0 = pl.multiple_of(base + c * CHP, 8)
            pltpu.async_copy(eis[g].at[1, pl.ds(r0, CHP), :], dstv[b],
                             semL[b])
            pltpu.async_copy(ews[g].at[pl.ds(r0, CHP), :], ewv[b], semL[b])

        def wait_ld(b):
            pltpu.make_async_copy(eis[g].at[1, pl.ds(0, CHP), :], dstv[b],
                                  semL[b]).wait()
            pltpu.make_async_copy(ews[g].at[pl.ds(0, CHP), :], ewv[b],
                                  semL[b]).wait()

        def fire_sc(b, jlo=0):
            for j in range(jlo, CHP):
                pltpu.async_copy(ewv[b].at[j], accs[g].at[dstv[b].at[j]],
                                 semS[b], add=True)

        def wait_sc(b, jlo=0):
            for j in range(jlo, CHP):
                pltpu.make_async_copy(ewv[b].at[j],
                                      accs[g].at[dstv[b].at[j]],
                                      semS[b]).wait()

        fire_ld(0, 0)
        fire_ld(1, 1)

        @pl.loop(0, npair)
        def _(p):
            c0 = 2 * p
            wait_ld(0)
            fire_sc(0)
            wait_sc(0)
            fire_ld(c0 + 2, 0)
            wait_ld(1)
            fire_sc(1)
            wait_sc(1)

            @pl.when(p < npair - 1)
            def _():
                fire_ld(c0 + 3, 1)

        wait_ld(0)
        fire_sc(0)
        wait_sc(0)

        @pl.when(jnp.logical_and(cid == 1, sid == NS - 1))
        def _():
            pltpu.sync_copy(eis[g].at[1, pl.ds(PART0, CHP), :], dstv[0])
            pltpu.sync_copy(ews[g].at[pl.ds(PART0, CHP), :], ewv[0])
            fire_sc(0, PARTJLO)
            wait_sc(0, PARTJLO)

    plsc.subcore_barrier()
    for g in range(4):
        off = pl.multiple_of((cid * 4 + g) * N_PAD + sid * NPT, 128)
        pltpu.sync_copy(accs[g].at[pl.ds(sid * NPT, NPT)],
                        out.at[pl.ds(off, NPT)])


@functools.partial(
    pl.kernel,
    out_type=jax.ShapeDtypeStruct((NC * 4 * N_PAD, H), jnp.float32),
    mesh=_mesh,
    compiler_params=pltpu.CompilerParams(use_tc_tiling_on_sc=False),
    scratch_types=[
        pltpu.VMEM((CHP, 128), jnp.int32),
        pltpu.VMEM((CHP, 128), jnp.int32),
        pltpu.VMEM((CHP, 128), jnp.float32),
        pltpu.VMEM((CHP, 128, H), jnp.float32),
        pltpu.VMEM((CHP, 128), jnp.int32),
        pltpu.VMEM((CHP, 128), jnp.int32),
        pltpu.VMEM((CHP, 128), jnp.float32),
        pltpu.VMEM((CHP, 128, H), jnp.float32),
        pltpu.VMEM((ZROW, H), jnp.float32),
        pltpu.VMEM_SHARED((N_PAD, H), jnp.float32),
        pltpu.SemaphoreType.DMA,
        pltpu.SemaphoreType.DMA,
        pltpu.SemaphoreType.DMA,
        pltpu.SemaphoreType.DMA,
        pltpu.SemaphoreType.DMA,
        pltpu.SemaphoreType.DMA,
    ],
)
def _prop_pass(y0, y1, y2, y3,
               e0, e1, e2, e3,
               w0, w1, w2, w3, out,
               srcv0, dstv0, ewv0, rows0,
               srcv1, dstv1, ewv1, rows1,
               zbuf, acc,
               semL0, semL1, semG0, semG1, semS0, semS1):
    cid = lax.axis_index("c")
    sid = lax.axis_index("s")
    ys = [y0, y1, y2, y3]
    eis = [e0, e1, e2, e3]
    ews = [w0, w1, w2, w3]
    srcv = [srcv0, srcv1]
    dstv = [dstv0, dstv1]
    ewv = [ewv0, ewv1]
    rows = [rows0, rows1]
    semL = [semL0, semL1]
    semG = [semG0, semG1]
    semS = [semS0, semS1]
    base, nch = _worker_plan(cid, sid)
    npair = (nch - 1) // 2
    _zero_fill(zbuf, ZROW, H)

    def zero_acc():
        for k in range(NPT // ZROW):
            pltpu.sync_copy(zbuf, acc.at[pl.ds(sid * NPT + k * ZROW, ZROW), :])

    zero_acc()
    plsc.subcore_barrier()

    for g in range(4):
        def fire_lin(c, b):
            r0 = pl.multiple_of(base + c * CHP, 8)
            pltpu.async_copy(eis[g].at[0, pl.ds(r0, CHP), :], srcv[b], semL[b])
            pltpu.async_copy(eis[g].at[1, pl.ds(r0, CHP), :], dstv[b], semL[b])
            pltpu.async_copy(ews[g].at[pl.ds(r0, CHP), :], ewv[b], semL[b])

        def wait_lin(b):
            pltpu.make_async_copy(eis[g].at[0, pl.ds(0, CHP), :], srcv[b],
                                  semL[b]).wait()
            pltpu.make_async_copy(eis[g].at[1, pl.ds(0, CHP), :], dstv[b],
                                  semL[b]).wait()
            pltpu.make_async_copy(ews[g].at[pl.ds(0, CHP), :], ewv[b],
                                  semL[b]).wait()

        def fire_gather(b, jlo=0):
            for j in range(jlo, CHP):
                pltpu.async_copy(ys[g].at[srcv[b].at[j]], rows[b].at[j],
                                 semG[b])

        def wait_gather(b, jlo=0):
            for j in range(jlo, CHP):
                pltpu.make_async_copy(ys[g].at[srcv[b].at[j]], rows[b].at[j],
                                      semG[b]).wait()

        def fire_scatter(b, jlo=0):
            for j in range(jlo, CHP):
                pltpu.async_copy(rows[b].at[j], acc.at[dstv[b].at[j]],
                                 semS[b], add=True)

        def wait_scatter(b, jlo=0):
            for j in range(jlo, CHP):
                pltpu.make_async_copy(rows[b].at[j], acc.at[dstv[b].at[j]],
                                      semS[b]).wait()

        def scale(b):
            @pl.loop(0, CHP)
            def _(r):
                for l0 in range(0, 128, 16):
                    ew16 = ewv[b][r, pl.ds(l0, 16)]
                    for k in range(16):
                        rows[b][r, l0 + k, :] = rows[b][r, l0 + k, :] * ew16[k]

        fire_lin(0, 0)
        fire_lin(1, 1)
        wait_lin(0)
        fire_gather(0)

        @pl.loop(0, npair)
        def _(p):
            c0 = 2 * p
            wait_lin(1)
            fire_gather(1)
            wait_gather(0)
            scale(0)
            fire_scatter(0)
            wait_scatter(0)
            fire_lin(c0 + 2, 0)
            wait_lin(0)
            fire_gather(0)
            wait_gather(1)
            scale(1)
            fire_scatter(1)
            wait_scatter(1)

            @pl.when(p < npair - 1)
            def _():
                fire_lin(c0 + 3, 1)

        wait_gather(0)
        scale(0)
        fire_scatter(0)
        wait_scatter(0)

        @pl.when(jnp.logical_and(cid == 1, sid == NS - 1))
        def _():
            pltpu.sync_copy(eis[g].at[0, pl.ds(PART0, CHP), :], srcv[0])
            pltpu.sync_copy(eis[g].at[1, pl.ds(PART0, CHP), :], dstv[0])
            pltpu.sync_copy(ews[g].at[pl.ds(PART0, CHP), :], ewv[0])
            fire_gather(0, PARTJLO)
            wait_gather(0, PARTJLO)
            scale(0)
            fire_scatter(0, PARTJLO)
            wait_scatter(0, PARTJLO)

        plsc.subcore_barrier()
        roff = pl.multiple_of((cid * 4 + g) * N_PAD + sid * NPT, 128)
        pltpu.sync_copy(acc.at[pl.ds(sid * NPT, NPT), :],
                        out.at[pl.ds(roff, NPT), :])
        if g < 3:
            zero_acc()
            plsc.subcore_barrier()


X8 = N_PAD // 8
XV = N // 8
DR = N_PAD // 128


def kernel(flatten, features, pfcnetworks, mdcbcnetworks, v1cnetworks,
           shanetworks, pfcnetworkweights, mdcbcnetworkweights,
           v1cnetworkweights, shanetworkweights,
           pfc_W1, pfc_b1, pfc_g, pfc_bt, pfc_W2, pfc_b2,
           mdcbc_W1, mdcbc_b1, mdcbc_g, mdcbc_bt, mdcbc_W2, mdcbc_b2,
           v1c_W1, v1c_b1, v1c_g, v1c_bt, v1c_W2, v1c_b2,
           sha_W1, sha_b1, sha_g, sha_bt, sha_W2, sha_b2,
           gate_W, gate_b):
    eis = [pfcnetworks, mdcbcnetworks, v1cnetworks, shanetworks]
    ews_in = [pfcnetworkweights, mdcbcnetworkweights, v1cnetworkweights,
              shanetworkweights]
    W1s = jnp.stack([pfc_W1, mdcbc_W1, v1c_W1, sha_W1])
    b1s = jnp.stack([pfc_b1, mdcbc_b1, v1c_b1, sha_b1])
    gs = jnp.stack([pfc_g, mdcbc_g, v1c_g, sha_g])
    bts = jnp.stack([pfc_bt, mdcbc_bt, v1c_bt, sha_bt])
    W2s = jnp.stack([pfc_W2, mdcbc_W2, v1c_W2, sha_W2])
    b2s = jnp.stack([pfc_b2, mdcbc_b2, v1c_b2, sha_b2])

    e3 = [ei.reshape(2, RT, 128) for ei in eis]
    w2d = [ew.reshape(RT, 128) for ew in ews_in]

    eye8 = jnp.eye(8, dtype=jnp.float32)
    w1bd = jnp.einsum("ab,gfh->gafbh", eye8, W1s).reshape(4, 120, 128)
    w1bd = jnp.pad(w1bd, ((0, 0), (0, 8), (0, 0)))
    w2p = jnp.pad(W2s, ((0, 0), (0, 0), (0, 14)))
    w2bd = jnp.einsum("ab,ghj->gahbj", eye8, w2p).reshape(4, 128, 128)
    b2t = jnp.tile(jnp.pad(b2s, ((0, 0), (0, 14))), (1, 8))
    b1t = jnp.tile(b1s, (1, 8))
    lane = jnp.arange(128)
    swap = jnp.where(lane % 16 == 0, lane + 1,
                     jnp.where(lane % 16 == 1, lane - 1, lane))
    perm = jax.nn.one_hot(swap, 128, dtype=jnp.float32)

    degp = _deg_pass(*e3, *w2d)
    degv = degp.reshape(NC * 4 * DR, 128)
    deg = (degv[:4 * DR] + degv[4 * DR:]).reshape(4, DR, 128) + 2.0
    dinv = jnp.where(deg > 0, lax.rsqrt(jnp.where(deg > 0, deg, 1.0)), 0.0)
    dinv16 = jnp.repeat(dinv, 16, axis=-1).reshape(4, X8, 128)

    fp = flatten.reshape(XV, 120)
    fp = jnp.pad(fp, ((0, X8 - XV), (0, 8)))
    h1 = jnp.einsum("xl,glo->gxo", fp, w1bd)
    y1 = dinv16 * h1

    zp1 = _prop_pass(*[y1[g].reshape(N_PAD, H) for g in range(4)],
                     *e3, *w2d)
    zv1 = zp1.reshape(NC * 4 * X8, 128)
    z1 = (zv1[:4 * X8] + zv1[4 * X8:]).reshape(4, X8, 128)
    a = jax.nn.relu(dinv16 * (z1 + 2.0 * y1) + b1t[:, None, :])
    s1 = jnp.sum(a[:, :XV], axis=1).reshape(4, 8, 16).sum(axis=1)
    s2 = jnp.sum(a[:, :XV] * a[:, :XV], axis=1).reshape(4, 8, 16).sum(axis=1)
    mu = s1 / N
    var = s2 / N - mu * mu
    alpha = gs * lax.rsqrt(var + 1e-5)
    beta = bts - mu * alpha
    alphat = jnp.tile(alpha, (1, 8))
    betat = jnp.tile(beta, (1, 8))
    y2 = dinv16 * (a * alphat[:, None, :] + betat[:, None, :])

    zp2 = _prop_pass(*[y2[g].reshape(N_PAD, H) for g in range(4)],
                     *e3, *w2d)
    zv2 = zp2.reshape(NC * 4 * X8, 128)
    z2 = (zv2[:4 * X8] + zv2[4 * X8:]).reshape(4, X8, 128)
    pre = dinv16 * (z2 + 2.0 * y2)
    logits = jnp.einsum("gxl,glo->gxo", pre, w2bd) + b2t[:, None, :]
    partner = jnp.einsum("gxl,lo->gxo", logits, perm)
    m = jnp.maximum(logits, partner)
    ls = logits - (m + jnp.log(jnp.exp(logits - m) + jnp.exp(partner - m)))

    glt = gate_W.T @ features.T + gate_b[:, None]
    wt = jax.nn.softmax(glt, axis=0)
    wtp = jnp.pad(wt, ((0, 0), (0, N_PAD - N))).reshape(4, DR, 128)
    wpk = jnp.repeat(wtp, 16, axis=-1).reshape(4, X8, 128)
    mixed = jnp.sum(wpk * ls, axis=0)
    return mixed.reshape(N_PAD, H)[:N, :2]


# speedup vs baseline: 72.0453x; 1.0569x over previous
import functools

import jax
import jax.numpy as jnp
from jax import lax
from jax.experimental import pallas as pl
from jax.experimental.pallas import tpu as pltpu
from jax.experimental.pallas import tpu_sc as plsc

N = 50000
E = 1600000
H = 16
NC = 2
NS = 16
NW = NC * NS
N_PAD = 51200
NPT = N_PAD // NS
RT = E // 128
PAIR_ROWS = 784
L0 = 392
L1 = PAIR_ROWS - L0
CHP = 8
ZROW = 400
LASTLEN = RT - (NS - 1) * PAIR_ROWS - L0
PART0 = RT - CHP
PARTJLO = CHP - (LASTLEN - (LASTLEN // CHP) * CHP)

_mesh = plsc.VectorSubcoreMesh(core_axis_name="c", subcore_axis_name="s")


def _zero_fill(zbuf, rows, width):
    @pl.loop(0, rows, unroll=8)
    def _(i):
        zbuf[i, :] = jnp.zeros((width,), jnp.float32)


def _worker_plan(cid, sid):
    base = sid * PAIR_ROWS + cid * L0
    nch = jnp.where(
        cid == 0, L0 // CHP,
        jnp.where(sid == NS - 1, LASTLEN // CHP, L1 // CHP))
    return base, nch


@functools.partial(
    pl.kernel,
    out_type=jax.ShapeDtypeStruct((NC * 4 * N_PAD,), jnp.float32),
    mesh=_mesh,
    compiler_params=pltpu.CompilerParams(use_tc_tiling_on_sc=False),
    scratch_types=[
        pltpu.VMEM((CHP, 128), jnp.int32),
        pltpu.VMEM((CHP, 128), jnp.float32),
        pltpu.VMEM((CHP, 128), jnp.int32),
        pltpu.VMEM((CHP, 128), jnp.float32),
        pltpu.VMEM((NPT,), jnp.float32),
        pltpu.VMEM_SHARED((N_PAD,), jnp.float32),
        pltpu.VMEM_SHARED((N_PAD,), jnp.float32),
        pltpu.VMEM_SHARED((N_PAD,), jnp.float32),
        pltpu.VMEM_SHARED((N_PAD,), jnp.float32),
        pltpu.SemaphoreType.DMA,
        pltpu.SemaphoreType.DMA,
        pltpu.SemaphoreType.DMA,
        pltpu.SemaphoreType.DMA,
    ],
)
def _deg_pass(e0, e1, e2, e3, w0, w1, w2, w3, out,
              dstv0, ewv0, dstv1, ewv1, zbuf, acc0, acc1, acc2, acc3,
              semL0, semL1, semS0, semS1):
    cid = lax.axis_index("c")
    sid = lax.axis_index("s")
    eis = [e0, e1, e2, e3]
    ews = [w0, w1, w2, w3]
    accs = [acc0, acc1, acc2, acc3]
    dstv = [dstv0, dstv1]
    ewv = [ewv0, ewv1]
    semL = [semL0, semL1]
    semS = [semS0, semS1]
    base, nch = _worker_plan(cid, sid)
    npair = (nch - 1) // 2

    @pl.loop(0, NPT // 16, unroll=8)
    def _(i):
        zbuf[pl.ds(i * 16, 16)] = jnp.zeros((16,), jnp.float32)

    for g in range(4):
        pltpu.sync_copy(zbuf, accs[g].at[pl.ds(sid * NPT, NPT)])
    plsc.subcore_barrier()

    for g in range(4):
        def fire_ld(c, b):
            r0 = pl.multiple_of(base + c * CHP, 8)
            pltpu.async_copy(eis[g].at[1, pl.ds(r0, CHP), :], dstv[b],
                             semL[b])
            pltpu.async_copy(ews[g].at[pl.ds(r0, CHP), :], ewv[b], semL[b])

        def wait_ld(b):
            pltpu.make_async_copy(eis[g].at[1, pl.ds(0, CHP), :], dstv[b],
                                  semL[b]).wait()
            pltpu.make_async_copy(ews[g].at[pl.ds(0, CHP), :], ewv[b],
                                  semL[b]).wait()

        def fire_sc(b, jlo=0):
            for j in range(jlo, CHP):
                pltpu.async_copy(ewv[b].at[j], accs[g].at[dstv[b].at[j]],
                                 semS[b], add=True)

        def wait_sc(b, jlo=0):
            for j in range(jlo, CHP):
                pltpu.make_async_copy(ewv[b].at[j],
                                      accs[g].at[dstv[b].at[j]],
                                      semS[b]).wait()

        fire_ld(0, 0)
        fire_ld(1, 1)

        @pl.loop(0, npair)
        def _(p):
            c0 = 2 * p
            wait_ld(0)
            fire_sc(0)
            wait_sc(0)
            fire_ld(c0 + 2, 0)
            wait_ld(1)
            fire_sc(1)
            wait_sc(1)

            @pl.when(p < npair - 1)
            def _():
                fire_ld(c0 + 3, 1)

        wait_ld(0)
        fire_sc(0)
        wait_sc(0)

        @pl.when(jnp.logical_and(cid == 1, sid == NS - 1))
        def _():
            pltpu.sync_copy(eis[g].at[1, pl.ds(PART0, CHP), :], dstv[0])
            pltpu.sync_copy(ews[g].at[pl.ds(PART0, CHP), :], ewv[0])
            fire_sc(0, PARTJLO)
            wait_sc(0, PARTJLO)

    plsc.subcore_barrier()
    for g in range(4):
        off = pl.multiple_of((cid * 4 + g) * N_PAD + sid * NPT, 128)
        pltpu.sync_copy(accs[g].at[pl.ds(sid * NPT, NPT)],
                        out.at[pl.ds(off, NPT)])


@functools.partial(
    pl.kernel,
    out_type=jax.ShapeDtypeStruct((NC * 4 * N_PAD, H), jnp.float32),
    mesh=_mesh,
    compiler_params=pltpu.CompilerParams(use_tc_tiling_on_sc=False),
    scratch_types=[
        pltpu.VMEM((CHP, 128), jnp.int32),
        pltpu.VMEM((CHP, 128), jnp.int32),
        pltpu.VMEM((CHP, 128), jnp.float32),
        pltpu.VMEM((CHP, 128, H), jnp.float32),
        pltpu.VMEM((CHP, 128), jnp.int32),
        pltpu.VMEM((CHP, 128), jnp.int32),
        pltpu.VMEM((CHP, 128), jnp.float32),
        pltpu.VMEM((CHP, 128, H), jnp.float32),
        pltpu.VMEM((ZROW, H), jnp.float32),
        pltpu.VMEM_SHARED((N_PAD, H), jnp.float32),
        pltpu.SemaphoreType.DMA,
        pltpu.SemaphoreType.DMA,
        pltpu.SemaphoreType.DMA,
        pltpu.SemaphoreType.DMA,
        pltpu.SemaphoreType.DMA,
        pltpu.SemaphoreType.DMA,
    ],
)
def _prop_pass(y0, y1, y2, y3,
               e0, e1, e2, e3,
               w0, w1, w2, w3, out,
               srcv0, dstv0, ewv0, rows0,
               srcv1, dstv1, ewv1, rows1,
               zbuf, acc,
               semL0, semL1, semG0, semG1, semS0, semS1):
    cid = lax.axis_index("c")
    sid = lax.axis_index("s")
    ys = [y0, y1, y2, y3]
    eis = [e0, e1, e2, e3]
    ews = [w0, w1, w2, w3]
    srcv = [srcv0, srcv1]
    dstv = [dstv0, dstv1]
    ewv = [ewv0, ewv1]
    rows = [rows0, rows1]
    semL = [semL0, semL1]
    semG = [semG0, semG1]
    semS = [semS0, semS1]
    base, nch = _worker_plan(cid, sid)
    npair = (nch - 1) // 2
    _zero_fill(zbuf, ZROW, H)

    def zero_acc():
        for k in range(NPT // ZROW):
            pltpu.sync_copy(zbuf, acc.at[pl.ds(sid * NPT + k * ZROW, ZROW), :])

    zero_acc()
    plsc.subcore_barrier()

    for g in range(4):
        def fire_lin(c, b):
            r0 = pl.multiple_of(base + c * CHP, 8)
            pltpu.async_copy(eis[g].at[0, pl.ds(r0, CHP), :], srcv[b], semL[b])
            pltpu.async_copy(eis[g].at[1, pl.ds(r0, CHP), :], dstv[b], semL[b])
            pltpu.async_copy(ews[g].at[pl.ds(r0, CHP), :], ewv[b], semL[b])

        def wait_lin(b):
            pltpu.make_async_copy(eis[g].at[0, pl.ds(0, CHP), :], srcv[b],
                                  semL[b]).wait()
            pltpu.make_async_copy(eis[g].at[1, pl.ds(0, CHP), :], dstv[b],
                                  semL[b]).wait()
            pltpu.make_async_copy(ews[g].at[pl.ds(0, CHP), :], ewv[b],
                                  semL[b]).wait()

        def fire_gather(b, jlo=0):
            for j in range(jlo, CHP):
                pltpu.async_copy(ys[g].at[srcv[b].at[j]], rows[b].at[j],
                                 semG[b])

        def wait_gather(b, jlo=0):
            for j in range(jlo, CHP):
                pltpu.make_async_copy(ys[g].at[srcv[b].at[j]], rows[b].at[j],
                                      semG[b]).wait()

        def fire_scatter(b, jlo=0):
            for j in range(jlo, CHP):
                pltpu.async_copy(rows[b].at[j], acc.at[dstv[b].at[j]],
                                 semS[b], add=True)

        def wait_scatter(b, jlo=0):
            for j in range(jlo, CHP):
                pltpu.make_async_copy(rows[b].at[j], acc.at[dstv[b].at[j]],
                                      semS[b]).wait()

        def scale(b):
            @pl.loop(0, CHP)
            def _(r):
                for l0 in range(0, 128, 16):
                    ew16 = ewv[b][r, pl.ds(l0, 16)]
                    for k in range(16):
                        rows[b][r, l0 + k, :] = rows[b][r, l0 + k, :] * ew16[k]

        fire_lin(0, 0)
        fire_lin(1, 1)
        wait_lin(0)
        fire_gather(0)

        @pl.loop(0, npair)
        def _(p):
            c0 = 2 * p
            wait_lin(1)
            fire_gather(1)
            wait_gather(0)
            scale(0)
            fire_scatter(0)
            wait_scatter(0)
            fire_lin(c0 + 2, 0)
            wait_lin(0)
            fire_gather(0)
            wait_gather(1)
            scale(1)
            fire_scatter(1)
            wait_scatter(1)

            @pl.when(p < npair - 1)
            def _():
                fire_lin(c0 + 3, 1)

        wait_gather(0)
        scale(0)
        fire_scatter(0)
        wait_scatter(0)

        @pl.when(jnp.logical_and(cid == 1, sid == NS - 1))
        def _():
            pltpu.sync_copy(eis[g].at[0, pl.ds(PART0, CHP), :], srcv[0])
            pltpu.sync_copy(eis[g].at[1, pl.ds(PART0, CHP), :], dstv[0])
            pltpu.sync_copy(ews[g].at[pl.ds(PART0, CHP), :], ewv[0])
            fire_gather(0, PARTJLO)
            wait_gather(0, PARTJLO)
            scale(0)
            fire_scatter(0, PARTJLO)
            wait_scatter(0, PARTJLO)

        plsc.subcore_barrier()
        roff = pl.multiple_of((cid * 4 + g) * N_PAD + sid * NPT, 128)
        pltpu.sync_copy(acc.at[pl.ds(sid * NPT, NPT), :],
                        out.at[pl.ds(roff, NPT), :])
        if g < 3:
            zero_acc()
            plsc.subcore_barrier()


X8 = N_PAD // 8
XV = N // 8
DR = N_PAD // 128


def kernel(flatten, features, pfcnetworks, mdcbcnetworks, v1cnetworks,
           shanetworks, pfcnetworkweights, mdcbcnetworkweights,
           v1cnetworkweights, shanetworkweights,
           pfc_W1, pfc_b1, pfc_g, pfc_bt, pfc_W2, pfc_b2,
           mdcbc_W1, mdcbc_b1, mdcbc_g, mdcbc_bt, mdcbc_W2, mdcbc_b2,
           v1c_W1, v1c_b1, v1c_g, v1c_bt, v1c_W2, v1c_b2,
           sha_W1, sha_b1, sha_g, sha_bt, sha_W2, sha_b2,
           gate_W, gate_b):
    eis = [pfcnetworks, mdcbcnetworks, v1cnetworks, shanetworks]
    ews_in = [pfcnetworkweights, mdcbcnetworkweights, v1cnetworkweights,
              shanetworkweights]
    W1s = jnp.stack([pfc_W1, mdcbc_W1, v1c_W1, sha_W1])
    b1s = jnp.stack([pfc_b1, mdcbc_b1, v1c_b1, sha_b1])
    gs = jnp.stack([pfc_g, mdcbc_g, v1c_g, sha_g])
    bts = jnp.stack([pfc_bt, mdcbc_bt, v1c_bt, sha_bt])
    W2s = jnp.stack([pfc_W2, mdcbc_W2, v1c_W2, sha_W2])
    b2s = jnp.stack([pfc_b2, mdcbc_b2, v1c_b2, sha_b2])

    e3 = [ei.reshape(2, RT, 128) for ei in eis]
    w2d = [ew.reshape(RT, 128) for ew in ews_in]

    eye8 = jnp.eye(8, dtype=jnp.float32)
    w1bd = jnp.einsum("ab,gfh->gafbh", eye8, W1s).reshape(4, 120, 128)
    w1bd = jnp.pad(w1bd, ((0, 0), (0, 8), (0, 0)))
    b1t = jnp.tile(b1s, (1, 8))
    kk = jnp.arange(16 * 128)
    ldst = 8 * (kk // 128) + (kk % 128) // 16
    hh = kk % 16
    oh = jax.nn.one_hot(ldst, 128, dtype=jnp.float32)
    w2nm = [oh[None] * W2s[:, hh, c][:, :, None] for c in range(2)]

    degp = _deg_pass(*e3, *w2d)
    degv = degp.reshape(NC * 4 * DR, 128)
    deg = (degv[:4 * DR] + degv[4 * DR:]).reshape(4, DR, 128) + 2.0
    dinv = jnp.where(deg > 0, lax.rsqrt(jnp.where(deg > 0, deg, 1.0)), 0.0)
    dinv16 = jnp.repeat(dinv, 16, axis=-1).reshape(4, X8, 128)

    fp = flatten.reshape(XV, 120)
    fp = jnp.pad(fp, ((0, X8 - XV), (0, 8)))
    h1 = jnp.einsum("xl,glo->gxo", fp, w1bd)
    y1 = dinv16 * h1

    zp1 = _prop_pass(*[y1[g].reshape(N_PAD, H) for g in range(4)],
                     *e3, *w2d)
    zv1 = zp1.reshape(NC * 4 * X8, 128)
    z1 = (zv1[:4 * X8] + zv1[4 * X8:]).reshape(4, X8, 128)
    a = jax.nn.relu(dinv16 * (z1 + 2.0 * y1) + b1t[:, None, :])
    s1 = jnp.sum(a[:, :XV], axis=1).reshape(4, 8, 16).sum(axis=1)
    s2 = jnp.sum(a[:, :XV] * a[:, :XV], axis=1).reshape(4, 8, 16).sum(axis=1)
    mu = s1 / N
    var = s2 / N - mu * mu
    alpha = gs * lax.rsqrt(var + 1e-5)
    beta = bts - mu * alpha
    alphat = jnp.tile(alpha, (1, 8))
    betat = jnp.tile(beta, (1, 8))
    y2 = dinv16 * (a * alphat[:, None, :] + betat[:, None, :])

    zp2 = _prop_pass(*[y2[g].reshape(N_PAD, H) for g in range(4)],
                     *e3, *w2d)
    zv2 = zp2.reshape(NC * 4 * X8, 128)
    z2 = (zv2[:4 * X8] + zv2[4 * X8:]).reshape(4, X8, 128)
    t2 = (z2 + 2.0 * y2).reshape(4, DR, 16 * 128)
    lg = [dinv * jnp.einsum("grk,gkl->grl", t2, w2nm[c])
          + b2s[:, c][:, None, None] for c in range(2)]
    m = jnp.maximum(lg[0], lg[1])
    lse = m + jnp.log(jnp.exp(lg[0] - m) + jnp.exp(lg[1] - m))

    glt = gate_W.T @ features.T + gate_b[:, None]
    wt = jax.nn.softmax(glt, axis=0)
    wtn = jnp.pad(wt, ((0, 0), (0, N_PAD - N))).reshape(4, DR, 128)
    m0 = jnp.sum(wtn * (lg[0] - lse), axis=0).reshape(N_PAD)
    m1 = jnp.sum(wtn * (lg[1] - lse), axis=0).reshape(N_PAD)
    return jnp.stack([m0[:N], m1[:N]], axis=1)
